# Initial kernel scaffold; baseline (speedup 1.0000x reference)
import jax
import jax.numpy as jnp
from jax.experimental import pallas as pl


def kernel(x_flat, W):
    raise NotImplementedError("write your pallas kernel here")


# speedup vs baseline: 5.4154x; 5.4154x over previous
import functools

import jax
import jax.numpy as jnp
from jax.experimental import pallas as pl
from jax.experimental.pallas import tpu as pltpu

N_TOK = 32768
DIM = 256
K = 256
BLK = 2048


def _body(x_ref, w_ref, o_ref):
    x = x_ref[...]
    w = w_ref[...]
    wt2 = jnp.sum(w * w, axis=0, keepdims=True)
    x2 = jnp.sum(x * x, axis=1, keepdims=True)
    cross = jax.lax.dot_general(
        x, w, (((1,), (0,)), ((), ())),
        preferred_element_type=jnp.float32,
    )
    dist = x2 + wt2 - 2.0 * cross
    m = jnp.min(dist, axis=1, keepdims=True)
    iota = jax.lax.broadcasted_iota(jnp.int32, dist.shape, 1)
    idx = jnp.min(jnp.where(dist == m, iota, K), axis=1, keepdims=True)
    onehot = (iota == idx).astype(jnp.float32)
    q = jax.lax.dot_general(
        onehot, w, (((1,), (0,)), ((), ())),
        preferred_element_type=jnp.float32,
    )
    o_ref[...] = x + (q - x)


@jax.jit
def kernel(x_flat, W):
    grid = (N_TOK // BLK,)
    return pl.pallas_call(
        _body,
        grid=grid,
        in_specs=[
            pl.BlockSpec((BLK, DIM), lambda i: (i, 0)),
            pl.BlockSpec((DIM, K), lambda i: (0, 0)),
        ],
        out_specs=pl.BlockSpec((BLK, DIM), lambda i: (i, 0)),
        out_shape=jax.ShapeDtypeStruct((N_TOK, DIM), jnp.float32),
    )(x_flat, W)
